# baseline (device time: 10115 ns/iter reference)
import jax
import jax.numpy as jnp
from jax import lax
from jax.experimental import pallas as pl
from jax.experimental.pallas import tpu as pltpu

N_DEV = 4
K = 8


def kernel(x):
    m_per, n = x.shape
    chunk = m_per // K

    def body(x_hbm, out_hbm, buf, acc_ref, local_ref, comm_ref,
             copy_sems, out_sem, send_sems, recv_sems):
        my_pos = lax.axis_index("i")

        def chunk_copy(i, slot):
            return pltpu.make_async_copy(
                x_hbm.at[pl.ds(i * chunk, chunk), :],
                buf.at[slot],
                copy_sems.at[slot],
            )

        chunk_copy(0, 0).start()
        for i in range(K):
            slot = i % 2
            if i + 1 < K:
                chunk_copy(i + 1, (i + 1) % 2).start()
            chunk_copy(i, slot).wait()
            csum = jnp.sum(buf[slot], axis=0, keepdims=True)
            if i == 0:
                acc_ref[:, :] = csum
            else:
                acc_ref[:, :] = acc_ref[:, :] + csum

        partial = acc_ref[:, :]
        local_ref[:, :] = partial
        comm_ref[pl.ds(my_pos, 1), :] = partial

        barrier_sem = pltpu.get_barrier_semaphore()
        for off in range(1, N_DEV):
            tgt = (my_pos + off) % N_DEV
            pl.semaphore_signal(
                barrier_sem, inc=1,
                device_id=(tgt,), device_id_type=pl.DeviceIdType.MESH,
            )
        pl.semaphore_wait(barrier_sem, N_DEV - 1)

        sends = []
        for off in range(1, N_DEV):
            tgt = (my_pos + off) % N_DEV
            rdma = pltpu.make_async_remote_copy(
                src_ref=local_ref,
                dst_ref=comm_ref.at[pl.ds(my_pos, 1)],
                send_sem=send_sems.at[off],
                recv_sem=recv_sems.at[my_pos],
                device_id=(tgt,),
                device_id_type=pl.DeviceIdType.MESH,
            )
            rdma.start()
            sends.append(rdma)

        for off in range(1, N_DEV):
            src = (my_pos + off) % N_DEV
            recv = pltpu.make_async_remote_copy(
                src_ref=local_ref,
                dst_ref=comm_ref.at[pl.ds(src, 1)],
                send_sem=send_sems.at[0],
                recv_sem=recv_sems.at[src],
                device_id=(src,),
                device_id_type=pl.DeviceIdType.MESH,
            )
            recv.wait_recv()

        local_ref[:, :] = jnp.sum(comm_ref[:, :], axis=0, keepdims=True)
        out_copy = pltpu.make_async_copy(local_ref, out_hbm, out_sem)
        out_copy.start()
        out_copy.wait()

        for rdma in sends:
            rdma.wait_send()

    return pl.pallas_call(
        body,
        out_shape=jax.ShapeDtypeStruct((1, n), jnp.float32),
        in_specs=[pl.BlockSpec(memory_space=pl.ANY)],
        out_specs=pl.BlockSpec(memory_space=pl.ANY),
        scratch_shapes=[
            pltpu.VMEM((2, chunk, n), jnp.float32),
            pltpu.VMEM((1, n), jnp.float32),
            pltpu.VMEM((1, n), jnp.float32),
            pltpu.VMEM((N_DEV, n), jnp.float32),
            pltpu.SemaphoreType.DMA((2,)),
            pltpu.SemaphoreType.DMA,
            pltpu.SemaphoreType.DMA((N_DEV,)),
            pltpu.SemaphoreType.DMA((N_DEV,)),
        ],
        compiler_params=pltpu.CompilerParams(collective_id=0),
    )(x)
